# baseline (speedup 1.0000x reference)
import jax
import jax.numpy as jnp
from jax import lax
from jax.experimental import pallas as pl
from jax.experimental.pallas import tpu as pltpu
from jax.experimental.pallas import tpu_sc as plsc

B = 16384
D = 784
NC, NS = 2, 16
RPW = D // NS
BH = B // NC
NG = BH // 16
RQ = 7
NQ = RPW // RQ


def _body(idx_hbm, tabt_hbm, out_hbm, tabt_v, idx_v, *bs):
    bufs, sems = bs[:RQ], bs[RQ:]
    rg = lax.axis_index("s")
    h = lax.axis_index("c")
    j0 = rg * RPW
    bbase = h * BH
    pltpu.sync_copy(tabt_hbm, tabt_v)
    pltpu.sync_copy(idx_hbm.at[pl.ds(bbase, BH)], idx_v)

    dnums = lax.GatherDimensionNumbers(
        offset_dims=(), collapsed_slice_dims=(0,), start_index_map=(0,)
    )

    def out_slice(j):
        return out_hbm.at[pl.ds(j * B + bbase, BH)]

    def quad(q, _):
        ja = j0 + RQ * q

        for rr in range(RQ):
            @pl.when(q > 0)
            def _():
                pltpu.make_async_copy(
                    bufs[rr], out_slice(ja - RQ + rr), sems[rr]
                ).wait()

        @plsc.parallel_loop(0, NG // 8, unroll=4)
        def per_g8(g8):
            xs = [idx_v[pl.ds((g8 * 8 + k) * 16, 16)] for k in range(8)]
            for rr in range(RQ):
                rowv = tabt_v[pl.ds((ja + rr) * 10, 16)]
                for k in range(8):
                    v = lax.gather(
                        rowv, xs[k][:, None], dnums, (1,),
                        mode=lax.GatherScatterMode.PROMISE_IN_BOUNDS,
                    )
                    bufs[rr][pl.ds((g8 * 8 + k) * 16, 16)] = v

        for rr in range(RQ):
            pltpu.async_copy(bufs[rr], out_slice(ja + rr), sems[rr])
        return 0

    lax.fori_loop(0, NQ, quad, 0)
    for rr in range(RQ):
        pltpu.make_async_copy(
            bufs[rr], out_slice(j0 + RQ * (NQ - 1) + rr), sems[rr]
        ).wait()


@jax.jit
def _gather_t(x, logits):
    mesh = plsc.VectorSubcoreMesh(core_axis_name="c", subcore_axis_name="s")
    idx = x.astype(jnp.int32)
    tabt = jnp.concatenate([logits.T.reshape(D * 10), jnp.zeros((16,), jnp.float32)])
    run = pl.kernel(
        _body,
        mesh=mesh,
        out_type=jax.ShapeDtypeStruct((D * B,), jnp.float32),
        scratch_types=(
            [pltpu.VMEM((D * 10 + 16,), jnp.float32), pltpu.VMEM((BH,), jnp.int32)]
            + [pltpu.VMEM((BH,), jnp.float32)] * RQ
            + [pltpu.SemaphoreType.DMA] * RQ
        ),
    )
    out_t = run(idx, tabt)
    return jnp.transpose(out_t.reshape(1, 28, 28, B), (3, 0, 1, 2))


def kernel(x, logits):
    return _gather_t(x, logits)


# speedup vs baseline: 1.0268x; 1.0268x over previous
import jax
import jax.numpy as jnp
from jax import lax
from jax.experimental import pallas as pl
from jax.experimental.pallas import tpu as pltpu
from jax.experimental.pallas import tpu_sc as plsc

B = 16384
D = 784
NC, NS = 2, 16
RPW = D // NS
BH = B // NC
NG = BH // 16
RQ = 7
NQ = RPW // RQ


def _body(idx_hbm, tabt_hbm, out_hbm, tabt_v, idx_v, *bs):
    bufs, sems = bs[:RQ], bs[RQ:]
    rg = lax.axis_index("s")
    h = lax.axis_index("c")
    j0 = rg * RPW
    bbase = h * BH
    pltpu.sync_copy(tabt_hbm, tabt_v)
    pltpu.sync_copy(idx_hbm.at[pl.ds(bbase, BH)], idx_v)

    dnums = lax.GatherDimensionNumbers(
        offset_dims=(), collapsed_slice_dims=(0,), start_index_map=(0,)
    )

    def out_slice(j):
        return out_hbm.at[pl.ds(j * B + bbase, BH)]

    def quad(q, _):
        ja = j0 + RQ * q

        for rr in range(RQ):
            @pl.when(q > 0)
            def _():
                pltpu.make_async_copy(
                    bufs[rr], out_slice(ja - RQ + rr), sems[rr]
                ).wait()

        @plsc.parallel_loop(0, NG // 8, unroll=2)
        def per_g8(g8):
            xs = [idx_v[pl.ds((g8 * 8 + k) * 16, 16)] for k in range(8)]
            for rr in range(RQ):
                rowv = tabt_v[pl.ds((ja + rr) * 10, 16)]
                for k in range(8):
                    v = lax.gather(
                        rowv, xs[k][:, None], dnums, (1,),
                        mode=lax.GatherScatterMode.PROMISE_IN_BOUNDS,
                    )
                    bufs[rr][pl.ds((g8 * 8 + k) * 16, 16)] = v

        for rr in range(RQ):
            pltpu.async_copy(bufs[rr], out_slice(ja + rr), sems[rr])
        return 0

    lax.fori_loop(0, NQ, quad, 0)
    for rr in range(RQ):
        pltpu.make_async_copy(
            bufs[rr], out_slice(j0 + RQ * (NQ - 1) + rr), sems[rr]
        ).wait()


@jax.jit
def _gather_t(x, logits):
    mesh = plsc.VectorSubcoreMesh(core_axis_name="c", subcore_axis_name="s")
    idx = x.astype(jnp.int32)
    tabt = jnp.concatenate([logits.T.reshape(D * 10), jnp.zeros((16,), jnp.float32)])
    run = pl.kernel(
        _body,
        mesh=mesh,
        out_type=jax.ShapeDtypeStruct((D * B,), jnp.float32),
        scratch_types=(
            [pltpu.VMEM((D * 10 + 16,), jnp.float32), pltpu.VMEM((BH,), jnp.int32)]
            + [pltpu.VMEM((BH,), jnp.float32)] * RQ
            + [pltpu.SemaphoreType.DMA] * RQ
        ),
    )
    out_t = run(idx, tabt)
    return jnp.transpose(out_t.reshape(1, 28, 28, B), (3, 0, 1, 2))


def kernel(x, logits):
    return _gather_t(x, logits)
